# baseline (speedup 1.0000x reference)
import jax
import jax.numpy as jnp
from jax.experimental import pallas as pl
from jax.experimental.pallas import tpu as pltpu

B = 1024
K = 100000
BB = 64
NBB = B // BB


def _body(s_ref, v_ref, o_ref):
    o_ref[...] = jnp.sum(s_ref[...] * v_ref[...], axis=1, keepdims=True)


def kernel(state, values):
    values_row = values.reshape(1, K)
    return pl.pallas_call(
        _body,
        grid=(NBB,),
        in_specs=[
            pl.BlockSpec((BB, K), lambda i: (i, 0)),
            pl.BlockSpec((1, K), lambda i: (0, 0)),
        ],
        out_specs=pl.BlockSpec((BB, 1), lambda i: (i, 0)),
        out_shape=jax.ShapeDtypeStruct((B, 1), jnp.float32),
        compiler_params=pltpu.CompilerParams(
            dimension_semantics=("parallel",),
        ),
    )(state, values_row)


# speedup vs baseline: 1.0117x; 1.0117x over previous
import jax
import jax.numpy as jnp
from jax.experimental import pallas as pl
from jax.experimental.pallas import tpu as pltpu

B = 1024
K = 100000
CH = 16
NCH = B // CH
NBUF = 6


def _body(s_hbm, v_ref, o_ref, *scratch):
    bufs = scratch[:NBUF]
    sems = scratch[NBUF]
    v = v_ref[...]

    def copy(i, slot):
        return pltpu.make_async_copy(
            s_hbm.at[pl.ds(i * CH, CH), :], bufs[slot], sems.at[slot])

    for slot in range(NBUF):
        copy(slot, slot).start()
    for i in range(NCH):
        slot = i % NBUF
        copy(i, slot).wait()
        o_ref[pl.ds(i * CH, CH), :] = jnp.sum(
            bufs[slot][...] * v, axis=1, keepdims=True)
        nxt = i + NBUF
        if nxt < NCH:
            copy(nxt, slot).start()


def kernel(state, values):
    values_row = values.reshape(1, K)
    return pl.pallas_call(
        _body,
        in_specs=[
            pl.BlockSpec(memory_space=pltpu.MemorySpace.HBM),
            pl.BlockSpec((1, K), lambda: (0, 0)),
        ],
        out_specs=pl.BlockSpec((B, 1), lambda: (0, 0)),
        out_shape=jax.ShapeDtypeStruct((B, 1), jnp.float32),
        scratch_shapes=(
            [pltpu.VMEM((CH, K), jnp.float32) for _ in range(NBUF)]
            + [pltpu.SemaphoreType.DMA((NBUF,))]
        ),
    )(state, values_row)
